# Initial kernel scaffold; baseline (speedup 1.0000x reference)
import jax
import jax.numpy as jnp
from jax.experimental import pallas as pl


def kernel(x, vectors):
    raise NotImplementedError("write your pallas kernel here")


# speedup vs baseline: 18.2154x; 18.2154x over previous
import jax
import jax.numpy as jnp
from jax import lax
from jax.experimental import pallas as pl
from jax.experimental.pallas import tpu as pltpu
from jax.experimental.pallas import tpu_sc as plsc

B = 16384
L = 50
D = 64
LANES = 16
NVREG = D // LANES

NC = 2
NS = 16
NW = NC * NS
SPW = B // NW
NBUF = 4


def _body(x_hbm, tab_hbm, out_hbm, idx_v, rows_v, out_v, *sems):
  wid = lax.axis_index("s") * NC + lax.axis_index("c")
  base = wid * SPW

  pltpu.sync_copy(x_hbm.at[pl.ds(base, SPW)], idx_v)

  def fire(s, slot):
    pltpu.async_copy(tab_hbm.at[idx_v.at[s]], rows_v.at[slot], sems[slot])

  def wait(s, slot):
    pltpu.make_async_copy(
        tab_hbm.at[idx_v.at[s]], rows_v.at[slot], sems[slot]).wait()

  lane = lax.iota(jnp.int32, LANES)
  last_one = jnp.where(lane == LANES - 1, 1.0, 0.0).astype(jnp.float32)
  zero = jnp.zeros((LANES,), jnp.float32)
  one = jnp.ones((LANES,), jnp.float32)

  def compute(s, slot):
    wait(s, slot)
    r = rows_v.at[slot]
    acc = [jnp.zeros((LANES,), jnp.float32) for _ in range(NVREG)]
    cnt = jnp.zeros((LANES,), jnp.float32)
    for l in range(L):
      regs = [r[l, pl.ds(k * LANES, LANES)] for k in range(NVREG)]
      t = (regs[0] + regs[1]) + (regs[2] + regs[3])
      for k in range(NVREG):
        acc[k] = acc[k] + regs[k]
      cs = plsc.cumsum(t)
      cnt = cnt + jnp.where(cs != 0.0, last_one, zero)
    tot = plsc.cummax(lax.rev(cnt, (0,)))
    inv = one / tot
    for k in range(NVREG):
      out_v[s, pl.ds(k * LANES, LANES)] = acc[k] * inv

  for b_ in range(NBUF):
    fire(b_, b_)

  def loop_body(g, carry):
    s0 = g * NBUF
    for b_ in range(NBUF):
      s = s0 + b_
      compute(s, b_)

      @pl.when(s + NBUF < SPW)
      def _():
        fire(s + NBUF, b_)

    return carry

  lax.fori_loop(0, SPW // NBUF, loop_body, 0)

  pltpu.sync_copy(out_v, out_hbm.at[pl.ds(base, SPW)])


@jax.jit
def kernel(x, vectors):
  mesh = plsc.VectorSubcoreMesh(core_axis_name="c", subcore_axis_name="s")
  run = pl.kernel(
      _body,
      out_type=jax.ShapeDtypeStruct((B, D), jnp.float32),
      mesh=mesh,
      compiler_params=pltpu.CompilerParams(
          needs_layout_passes=False, use_tc_tiling_on_sc=False),
      scratch_types=[
          pltpu.VMEM((SPW, L), jnp.int32),
          pltpu.VMEM((NBUF, L, D), jnp.float32),
          pltpu.VMEM((SPW, D), jnp.float32),
      ] + [pltpu.SemaphoreType.DMA] * NBUF,
  )
  return run(x, vectors)
